# Initial kernel scaffold; baseline (speedup 1.0000x reference)
import jax
import jax.numpy as jnp
from jax.experimental import pallas as pl


def kernel(x_ncthw, stem_wm, stem_s, stem_b, s2_b0_a_wm, s2_b0_a_s, s2_b0_a_b, s2_b0_b_wm, s2_b0_b_s, s2_b0_b_b, s2_b0_c_wm, s2_b0_c_s, s2_b0_c_b, s2_b0_proj_wm, s2_b0_proj_s, s2_b0_proj_b, s2_b1_a_wm, s2_b1_a_s, s2_b1_a_b, s2_b1_b_wm, s2_b1_b_s, s2_b1_b_b, s2_b1_c_wm, s2_b1_c_s, s2_b1_c_b, s3_b0_a_wm, s3_b0_a_s, s3_b0_a_b, s3_b0_b_wm, s3_b0_b_s, s3_b0_b_b, s3_b0_c_wm, s3_b0_c_s, s3_b0_c_b, s3_b0_proj_wm, s3_b0_proj_s, s3_b0_proj_b, s3_b1_a_wm, s3_b1_a_s, s3_b1_a_b, s3_b1_b_wm, s3_b1_b_s, s3_b1_b_b, s3_b1_c_wm, s3_b1_c_s, s3_b1_c_b, s4_b0_a_wm, s4_b0_a_s, s4_b0_a_b, s4_b0_b_wm, s4_b0_b_s, s4_b0_b_b, s4_b0_c_wm, s4_b0_c_s, s4_b0_c_b, s4_b0_proj_wm, s4_b0_proj_s, s4_b0_proj_b, s4_b1_a_wm, s4_b1_a_s, s4_b1_a_b, s4_b1_b_wm, s4_b1_b_s, s4_b1_b_b, s4_b1_c_wm, s4_b1_c_s, s4_b1_c_b, s5_b0_a_wm, s5_b0_a_s, s5_b0_a_b, s5_b0_b_wm, s5_b0_b_s, s5_b0_b_b, s5_b0_c_wm, s5_b0_c_s, s5_b0_c_b, s5_b0_proj_wm, s5_b0_proj_s, s5_b0_proj_b, s5_b1_a_wm, s5_b1_a_s, s5_b1_a_b, s5_b1_b_wm, s5_b1_b_s, s5_b1_b_b, s5_b1_c_wm, s5_b1_c_s, s5_b1_c_b, head_w, head_s, head_b):
    raise NotImplementedError("write your pallas kernel here")


# speedup vs baseline: 131.9369x; 131.9369x over previous
import numpy as np

import jax
import jax.numpy as jnp
from jax.experimental import pallas as pl
from jax.experimental.pallas import tpu as pltpu

_B1 = 8
_B2 = 2


def _stem_kernel(x_ref, wb_ref, s_ref, b_ref, o_ref):
    x = x_ref[...]
    wb = wb_ref[...]
    bb = x.shape[0]
    acc = None
    for dh in range(7):
        xh = x[:, dh:dh + 56]
        xh = xh.reshape(bb, 28, 2, 12, 186)[:, :, 0]
        for dt in range(5):
            xt = xh[:, :, dt:dt + 8, :]
            idx = dt * 7 + dh
            y = jnp.dot(xt.reshape(bb * 28 * 8, 186),
                        wb[idx * 186:(idx + 1) * 186, :],
                        preferred_element_type=jnp.float32)
            acc = y if acc is None else acc + y
    y = jnp.maximum(acc * s_ref[...] + b_ref[...], 0.0)
    o_ref[...] = y.reshape(bb, 28, 8, 224)


def _run_stem(x_ncthw, stem_wm, stem_s, stem_b):
    n = x_ncthw.shape[0]
    x5 = jnp.transpose(x_ncthw, (0, 3, 2, 1, 4)).astype(jnp.float32)
    xp = jnp.pad(x5, ((0, 0), (3, 3), (2, 2), (0, 0), (3, 3)))
    xp = xp.reshape(n, 62, 12, 3 * 62)

    wt = stem_wm.reshape(5, 7, 7, 3, 8)
    sel = np.zeros((7, 62, 28), np.float32)
    for dw in range(7):
        for wo in range(28):
            sel[dw, 2 * wo + dw, wo] = 1.0
    wb = jnp.einsum('dxy,thdcu->thcxyu', jnp.asarray(sel), wt)
    wb = wb.reshape(35 * 186, 224)

    s224 = jnp.tile(stem_s, (1, 28))
    b224 = jnp.tile(stem_b, (1, 28))

    return pl.pallas_call(
        _stem_kernel,
        out_shape=jax.ShapeDtypeStruct((n, 28, 8, 224), jnp.float32),
        grid=(n // _B1,),
        in_specs=[
            pl.BlockSpec((_B1, 62, 12, 186), lambda i: (i, 0, 0, 0)),
            pl.BlockSpec(wb.shape, lambda i: (0, 0)),
            pl.BlockSpec(s224.shape, lambda i: (0, 0)),
            pl.BlockSpec(b224.shape, lambda i: (0, 0)),
        ],
        out_specs=pl.BlockSpec((_B1, 28, 8, 224), lambda i: (i, 0, 0, 0)),
        compiler_params=pltpu.CompilerParams(
            dimension_semantics=("parallel",),
            vmem_limit_bytes=56 * 1024 * 1024),
    )(xp, wb, s224, b224)


def _cat_pad(x, axis, lo, hi, value):
    parts = []
    if lo:
        sh = list(x.shape)
        sh[axis] = lo
        parts.append(jnp.full(sh, value, x.dtype))
    parts.append(x)
    if hi:
        sh = list(x.shape)
        sh[axis] = hi
        parts.append(jnp.full(sh, value, x.dtype))
    return jnp.concatenate(parts, axis=axis) if len(parts) > 1 else x


def _pad_hw(x, hpad, wpad, value=0.0):
    if hpad:
        x = _cat_pad(x, 1, hpad, hpad, value)
    if wpad:
        x = _cat_pad(x, 2, wpad, wpad, value)
    return x


def _ds_h(x, ho, s):
    b, hs, w, t, c = x.shape
    if s == 1:
        return x[:, :ho]
    need = ho * s
    if hs < need:
        x = _cat_pad(x, 1, 0, need - hs, 0.0)
    elif hs > need:
        x = x[:, :need]
    x = x.reshape(b, ho, s, w * t, c)
    x = x[:, :, 0]
    return x.reshape(b, ho, w, t, c)


def _ds_w(x, wo, s):
    b, h, ws, t, c = x.shape
    if s == 1:
        return x[:, :, :wo]
    need = wo * s
    if ws < need:
        x = _cat_pad(x, 2, 0, need - ws, 0.0)
    elif ws > need:
        x = x[:, :, :need]
    x = x.reshape(b * h, wo, s, t, c)
    x = x[:, :, 0]
    return x.reshape(b, h, wo, t, c)


def _mm_bn(x, wm, s, b, relu, extra=None):
    lead = x.shape[:-1]
    m = 1
    for d in lead:
        m *= d
    y = jnp.dot(x.reshape(m, x.shape[-1]), wm,
                preferred_element_type=jnp.float32)
    y = y * s + b
    if extra is not None:
        y = y + extra.reshape(m, wm.shape[1])
    if relu:
        y = jnp.maximum(y, 0.0)
    return y.reshape(lead + (wm.shape[1],))


def _conv_t(x, wm, s, b, k):
    if k == 1:
        return _mm_bn(x, wm, s, b, True)
    t = x.shape[3]
    xp = _cat_pad(x, 3, k // 2, k // 2, 0.0)
    taps = [xp[:, :, :, dt:dt + t, :] for dt in range(k)]
    xc = jnp.concatenate(taps, axis=-1)
    return _mm_bn(xc, wm, s, b, True)


def _conv_s(x, wm, s, b, stride):
    h, w = x.shape[1], x.shape[2]
    xp = _pad_hw(x, 1, 1)
    ho = (h - 1) // stride + 1
    wo = (w - 1) // stride + 1
    taps = []
    for dh in range(3):
        xh = _ds_h(xp[:, dh:], ho, stride)
        for dw in range(3):
            taps.append(_ds_w(xh[:, :, dw:], wo, stride))
    xc = jnp.concatenate(taps, axis=-1)
    return _mm_bn(xc, wm, s, b, True)


def _maxpool_133(x):
    h, w = x.shape[1], x.shape[2]
    xp = _pad_hw(x, 1, 1, value=-jnp.inf)
    ho = (h - 1) // 2 + 1
    wo = (w - 1) // 2 + 1
    m = None
    for dh in range(3):
        xh = _ds_h(xp[:, dh:], ho, 2)
        for dw in range(3):
            tap = _ds_w(xh[:, :, dw:], wo, 2)
            m = tap if m is None else jnp.maximum(m, tap)
    return m


def _res_block(x, pa, pb, pc, pproj, temp_k, stride):
    ya = _conv_t(x, pa[0], pa[1], pa[2], temp_k)
    yb = _conv_s(ya, pb[0], pb[1], pb[2], stride)
    if pproj is None:
        shortcut = x
    else:
        xs = x
        if stride != 1:
            ho = (x.shape[1] - 1) // stride + 1
            wo = (x.shape[2] - 1) // stride + 1
            xs = _ds_w(_ds_h(x, ho, stride), wo, stride)
        shortcut = _mm_bn(xs, pproj[0], pproj[1], pproj[2], False)
    return _mm_bn(yb, pc[0], pc[1], pc[2], True, extra=shortcut)


_STAGES = (("s2", 3, 1), ("s3", 1, 2), ("s4", 1, 2), ("s5", 3, 2))


def _stages_kernel(*refs):
    o_ref = refs[-1]
    vals = [r[...] for r in refs[:-1]]
    x = vals[0]
    p = vals[1:]

    x = _maxpool_133(x)
    i = 0
    for _, temp_k, stride in _STAGES:
        pa = p[i:i + 3]
        pb = p[i + 3:i + 6]
        pc = p[i + 6:i + 9]
        pproj = p[i + 9:i + 12]
        i += 12
        x = _res_block(x, pa, pb, pc, pproj, temp_k, stride)
        pa = p[i:i + 3]
        pb = p[i + 3:i + 6]
        pc = p[i + 6:i + 9]
        i += 9
        x = _res_block(x, pa, pb, pc, None, temp_k, 1)

    bb, h, w, t, c = x.shape
    feat = jnp.mean(x.reshape(bb, h * w * t, c), axis=1)
    hw, hs, hb = p[i], p[i + 1], p[i + 2]
    y = jnp.dot(feat, hw, preferred_element_type=jnp.float32)
    y = y * hs + hb
    o_ref[...] = y.reshape(1, bb, hw.shape[1])


def kernel(x_ncthw, stem_wm, stem_s, stem_b, s2_b0_a_wm, s2_b0_a_s, s2_b0_a_b, s2_b0_b_wm, s2_b0_b_s, s2_b0_b_b, s2_b0_c_wm, s2_b0_c_s, s2_b0_c_b, s2_b0_proj_wm, s2_b0_proj_s, s2_b0_proj_b, s2_b1_a_wm, s2_b1_a_s, s2_b1_a_b, s2_b1_b_wm, s2_b1_b_s, s2_b1_b_b, s2_b1_c_wm, s2_b1_c_s, s2_b1_c_b, s3_b0_a_wm, s3_b0_a_s, s3_b0_a_b, s3_b0_b_wm, s3_b0_b_s, s3_b0_b_b, s3_b0_c_wm, s3_b0_c_s, s3_b0_c_b, s3_b0_proj_wm, s3_b0_proj_s, s3_b0_proj_b, s3_b1_a_wm, s3_b1_a_s, s3_b1_a_b, s3_b1_b_wm, s3_b1_b_s, s3_b1_b_b, s3_b1_c_wm, s3_b1_c_s, s3_b1_c_b, s4_b0_a_wm, s4_b0_a_s, s4_b0_a_b, s4_b0_b_wm, s4_b0_b_s, s4_b0_b_b, s4_b0_c_wm, s4_b0_c_s, s4_b0_c_b, s4_b0_proj_wm, s4_b0_proj_s, s4_b0_proj_b, s4_b1_a_wm, s4_b1_a_s, s4_b1_a_b, s4_b1_b_wm, s4_b1_b_s, s4_b1_b_b, s4_b1_c_wm, s4_b1_c_s, s4_b1_c_b, s5_b0_a_wm, s5_b0_a_s, s5_b0_a_b, s5_b0_b_wm, s5_b0_b_s, s5_b0_b_b, s5_b0_c_wm, s5_b0_c_s, s5_b0_c_b, s5_b0_proj_wm, s5_b0_proj_s, s5_b0_proj_b, s5_b1_a_wm, s5_b1_a_s, s5_b1_a_b, s5_b1_b_wm, s5_b1_b_s, s5_b1_b_b, s5_b1_c_wm, s5_b1_c_s, s5_b1_c_b, head_w, head_s, head_b):
    n = x_ncthw.shape[0]
    ncls = head_w.shape[1]

    y1 = _run_stem(x_ncthw, stem_wm, stem_s, stem_b)
    h1 = jnp.transpose(y1.reshape(n, 28, 8, 28, 8), (0, 1, 3, 2, 4))

    args = (s2_b0_a_wm, s2_b0_a_s, s2_b0_a_b, s2_b0_b_wm, s2_b0_b_s, s2_b0_b_b, s2_b0_c_wm, s2_b0_c_s, s2_b0_c_b, s2_b0_proj_wm, s2_b0_proj_s, s2_b0_proj_b, s2_b1_a_wm, s2_b1_a_s, s2_b1_a_b, s2_b1_b_wm, s2_b1_b_s, s2_b1_b_b, s2_b1_c_wm, s2_b1_c_s, s2_b1_c_b, s3_b0_a_wm, s3_b0_a_s, s3_b0_a_b, s3_b0_b_wm, s3_b0_b_s, s3_b0_b_b, s3_b0_c_wm, s3_b0_c_s, s3_b0_c_b, s3_b0_proj_wm, s3_b0_proj_s, s3_b0_proj_b, s3_b1_a_wm, s3_b1_a_s, s3_b1_a_b, s3_b1_b_wm, s3_b1_b_s, s3_b1_b_b, s3_b1_c_wm, s3_b1_c_s, s3_b1_c_b, s4_b0_a_wm, s4_b0_a_s, s4_b0_a_b, s4_b0_b_wm, s4_b0_b_s, s4_b0_b_b, s4_b0_c_wm, s4_b0_c_s, s4_b0_c_b, s4_b0_proj_wm, s4_b0_proj_s, s4_b0_proj_b, s4_b1_a_wm, s4_b1_a_s, s4_b1_a_b, s4_b1_b_wm, s4_b1_b_s, s4_b1_b_b, s4_b1_c_wm, s4_b1_c_s, s4_b1_c_b, s5_b0_a_wm, s5_b0_a_s, s5_b0_a_b, s5_b0_b_wm, s5_b0_b_s, s5_b0_b_b, s5_b0_c_wm, s5_b0_c_s, s5_b0_c_b, s5_b0_proj_wm, s5_b0_proj_s, s5_b0_proj_b, s5_b1_a_wm, s5_b1_a_s, s5_b1_a_b, s5_b1_b_wm, s5_b1_b_s, s5_b1_b_b, s5_b1_c_wm, s5_b1_c_s, s5_b1_c_b, head_w, head_s, head_b)

    x_spec = pl.BlockSpec((_B2, 28, 28, 8, 8), lambda i: (i, 0, 0, 0, 0))
    w_specs = [pl.BlockSpec(a.shape, lambda i, _nd=a.ndim: (0,) * _nd)
               for a in args]

    out = pl.pallas_call(
        _stages_kernel,
        out_shape=jax.ShapeDtypeStruct((n // _B2, _B2, ncls), jnp.float32),
        grid=(n // _B2,),
        in_specs=[x_spec] + w_specs,
        out_specs=pl.BlockSpec((1, _B2, ncls), lambda i: (i, 0, 0)),
        compiler_params=pltpu.CompilerParams(
            dimension_semantics=("parallel",),
            vmem_limit_bytes=56 * 1024 * 1024),
    )(h1, *args)
    return out.reshape(n, ncls)


# speedup vs baseline: 149.5432x; 1.1334x over previous
import numpy as np

import jax
import jax.numpy as jnp
from jax.experimental import pallas as pl
from jax.experimental.pallas import tpu as pltpu

_B1 = 8
_B2 = 4


def _stem_kernel(x_ref, wb_ref, s_ref, b_ref, o_ref):
    x = x_ref[...]
    wb = wb_ref[...]
    bb = x.shape[0]
    zh = jnp.zeros((bb, 3, 8, 168), x.dtype)
    x = jnp.concatenate([zh, x, zh], axis=1)
    zt = jnp.zeros((bb, 62, 2, 168), x.dtype)
    x = jnp.concatenate([zt, x, zt], axis=2)
    x = x.astype(jnp.bfloat16)
    acc = None
    for dh in range(7):
        xh = x[:, dh:dh + 56]
        xh = xh.reshape(bb, 28, 2, 12, 168)[:, :, 0]
        for dt in range(5):
            xt = xh[:, :, dt:dt + 8, :]
            idx = dt * 7 + dh
            y = jnp.dot(xt.reshape(bb * 28 * 8, 168),
                        wb[idx * 168:(idx + 1) * 168, :],
                        preferred_element_type=jnp.float32)
            acc = y if acc is None else acc + y
    y = jnp.maximum(acc * s_ref[...] + b_ref[...], 0.0)
    o_ref[...] = y.reshape(bb, 28, 8, 224)


def _run_stem(x_ncthw, stem_wm, stem_s, stem_b):
    n = x_ncthw.shape[0]
    x5 = jnp.transpose(x_ncthw, (0, 3, 2, 1, 4)).astype(jnp.float32)
    xp = x5.reshape(n, 56, 8, 3 * 56)

    wt = stem_wm.reshape(5, 7, 7, 3, 8)
    sel = np.zeros((7, 56, 28), np.float32)
    for dw in range(7):
        for wo in range(28):
            wi = 2 * wo + dw - 3
            if 0 <= wi < 56:
                sel[dw, wi, wo] = 1.0
    wb = jnp.einsum('dxy,thdcu->thcxyu', jnp.asarray(sel), wt)
    wb = wb.reshape(35 * 168, 224).astype(jnp.bfloat16)

    s224 = jnp.tile(stem_s, (1, 28))
    b224 = jnp.tile(stem_b, (1, 28))

    return pl.pallas_call(
        _stem_kernel,
        out_shape=jax.ShapeDtypeStruct((n, 28, 8, 224), jnp.float32),
        grid=(n // _B1,),
        in_specs=[
            pl.BlockSpec((_B1, 56, 8, 168), lambda i: (i, 0, 0, 0)),
            pl.BlockSpec(wb.shape, lambda i: (0, 0)),
            pl.BlockSpec(s224.shape, lambda i: (0, 0)),
            pl.BlockSpec(b224.shape, lambda i: (0, 0)),
        ],
        out_specs=pl.BlockSpec((_B1, 28, 8, 224), lambda i: (i, 0, 0, 0)),
        compiler_params=pltpu.CompilerParams(
            dimension_semantics=("parallel",),
            vmem_limit_bytes=56 * 1024 * 1024),
    )(xp, wb, s224, b224)


def _cat_pad(x, axis, lo, hi, value):
    parts = []
    if lo:
        sh = list(x.shape)
        sh[axis] = lo
        parts.append(jnp.full(sh, value, x.dtype))
    parts.append(x)
    if hi:
        sh = list(x.shape)
        sh[axis] = hi
        parts.append(jnp.full(sh, value, x.dtype))
    return jnp.concatenate(parts, axis=axis) if len(parts) > 1 else x


def _pad_hw(x, hpad, wpad, value=0.0):
    if hpad:
        x = _cat_pad(x, 1, hpad, hpad, value)
    if wpad:
        x = _cat_pad(x, 2, wpad, wpad, value)
    return x


def _ds_h(x, ho, s):
    b, hs, w, t, c = x.shape
    if s == 1:
        return x[:, :ho]
    need = ho * s
    if hs < need:
        x = _cat_pad(x, 1, 0, need - hs, 0.0)
    elif hs > need:
        x = x[:, :need]
    x = x.reshape(b, ho, s, w * t, c)
    x = x[:, :, 0]
    return x.reshape(b, ho, w, t, c)


def _ds_w(x, wo, s):
    b, h, ws, t, c = x.shape
    if s == 1:
        return x[:, :, :wo]
    need = wo * s
    if ws < need:
        x = _cat_pad(x, 2, 0, need - ws, 0.0)
    elif ws > need:
        x = x[:, :, :need]
    x = x.reshape(b * h, wo, s, t, c)
    x = x[:, :, 0]
    return x.reshape(b, h, wo, t, c)


def _mm_bn(x, wm, s, b, relu, extra=None):
    lead = x.shape[:-1]
    m = 1
    for d in lead:
        m *= d
    y = jnp.dot(x.reshape(m, x.shape[-1]), wm,
                preferred_element_type=jnp.float32)
    y = y * s + b
    if extra is not None:
        y = y + extra.reshape(m, wm.shape[1])
    if relu:
        y = jnp.maximum(y, 0.0)
    return y.reshape(lead + (wm.shape[1],))


def _conv_t(x, wm, s, b, k):
    if k == 1:
        return _mm_bn(x, wm, s, b, True)
    t = x.shape[3]
    xp = _cat_pad(x, 3, k // 2, k // 2, 0.0)
    taps = [xp[:, :, :, dt:dt + t, :] for dt in range(k)]
    xc = jnp.concatenate(taps, axis=-1)
    return _mm_bn(xc, wm, s, b, True)


def _conv_s(x, wm, s, b, stride):
    h, w = x.shape[1], x.shape[2]
    xp = _pad_hw(x, 1, 1)
    ho = (h - 1) // stride + 1
    wo = (w - 1) // stride + 1
    taps = []
    for dh in range(3):
        xh = _ds_h(xp[:, dh:], ho, stride)
        for dw in range(3):
            taps.append(_ds_w(xh[:, :, dw:], wo, stride))
    xc = jnp.concatenate(taps, axis=-1)
    return _mm_bn(xc, wm, s, b, True)


def _maxpool_133(x):
    h, w = x.shape[1], x.shape[2]
    xp = _pad_hw(x, 1, 1, value=-jnp.inf)
    ho = (h - 1) // 2 + 1
    wo = (w - 1) // 2 + 1
    m = None
    for dh in range(3):
        xh = _ds_h(xp[:, dh:], ho, 2)
        for dw in range(3):
            tap = _ds_w(xh[:, :, dw:], wo, 2)
            m = tap if m is None else jnp.maximum(m, tap)
    return m


def _res_block(x, pa, pb, pc, pproj, temp_k, stride):
    ya = _conv_t(x, pa[0], pa[1], pa[2], temp_k)
    yb = _conv_s(ya, pb[0], pb[1], pb[2], stride)
    if pproj is None:
        shortcut = x
    else:
        xs = x
        if stride != 1:
            ho = (x.shape[1] - 1) // stride + 1
            wo = (x.shape[2] - 1) // stride + 1
            xs = _ds_w(_ds_h(x, ho, stride), wo, stride)
        shortcut = _mm_bn(xs, pproj[0], pproj[1], pproj[2], False)
    return _mm_bn(yb, pc[0], pc[1], pc[2], True, extra=shortcut)


_STAGES = (("s2", 3, 1), ("s3", 1, 2), ("s4", 1, 2), ("s5", 3, 2))


def _stages_kernel(*refs):
    o_ref = refs[-1]
    vals = [r[...] for r in refs[:-1]]
    x = vals[0]
    p = vals[1:]

    x = _maxpool_133(x)
    i = 0
    for _, temp_k, stride in _STAGES:
        pa = p[i:i + 3]
        pb = p[i + 3:i + 6]
        pc = p[i + 6:i + 9]
        pproj = p[i + 9:i + 12]
        i += 12
        x = _res_block(x, pa, pb, pc, pproj, temp_k, stride)
        pa = p[i:i + 3]
        pb = p[i + 3:i + 6]
        pc = p[i + 6:i + 9]
        i += 9
        x = _res_block(x, pa, pb, pc, None, temp_k, 1)

    bb, h, w, t, c = x.shape
    feat = jnp.mean(x.reshape(bb, h * w * t, c), axis=1)
    hw, hs, hb = p[i], p[i + 1], p[i + 2]
    y = jnp.dot(feat, hw, preferred_element_type=jnp.float32)
    y = y * hs + hb
    o_ref[...] = y.reshape(1, bb, hw.shape[1])


def kernel(x_ncthw, stem_wm, stem_s, stem_b, s2_b0_a_wm, s2_b0_a_s, s2_b0_a_b, s2_b0_b_wm, s2_b0_b_s, s2_b0_b_b, s2_b0_c_wm, s2_b0_c_s, s2_b0_c_b, s2_b0_proj_wm, s2_b0_proj_s, s2_b0_proj_b, s2_b1_a_wm, s2_b1_a_s, s2_b1_a_b, s2_b1_b_wm, s2_b1_b_s, s2_b1_b_b, s2_b1_c_wm, s2_b1_c_s, s2_b1_c_b, s3_b0_a_wm, s3_b0_a_s, s3_b0_a_b, s3_b0_b_wm, s3_b0_b_s, s3_b0_b_b, s3_b0_c_wm, s3_b0_c_s, s3_b0_c_b, s3_b0_proj_wm, s3_b0_proj_s, s3_b0_proj_b, s3_b1_a_wm, s3_b1_a_s, s3_b1_a_b, s3_b1_b_wm, s3_b1_b_s, s3_b1_b_b, s3_b1_c_wm, s3_b1_c_s, s3_b1_c_b, s4_b0_a_wm, s4_b0_a_s, s4_b0_a_b, s4_b0_b_wm, s4_b0_b_s, s4_b0_b_b, s4_b0_c_wm, s4_b0_c_s, s4_b0_c_b, s4_b0_proj_wm, s4_b0_proj_s, s4_b0_proj_b, s4_b1_a_wm, s4_b1_a_s, s4_b1_a_b, s4_b1_b_wm, s4_b1_b_s, s4_b1_b_b, s4_b1_c_wm, s4_b1_c_s, s4_b1_c_b, s5_b0_a_wm, s5_b0_a_s, s5_b0_a_b, s5_b0_b_wm, s5_b0_b_s, s5_b0_b_b, s5_b0_c_wm, s5_b0_c_s, s5_b0_c_b, s5_b0_proj_wm, s5_b0_proj_s, s5_b0_proj_b, s5_b1_a_wm, s5_b1_a_s, s5_b1_a_b, s5_b1_b_wm, s5_b1_b_s, s5_b1_b_b, s5_b1_c_wm, s5_b1_c_s, s5_b1_c_b, head_w, head_s, head_b):
    n = x_ncthw.shape[0]
    ncls = head_w.shape[1]

    y1 = _run_stem(x_ncthw, stem_wm, stem_s, stem_b)
    h1 = jnp.transpose(y1.reshape(n, 28, 8, 28, 8), (0, 1, 3, 2, 4))

    args = (s2_b0_a_wm, s2_b0_a_s, s2_b0_a_b, s2_b0_b_wm, s2_b0_b_s, s2_b0_b_b, s2_b0_c_wm, s2_b0_c_s, s2_b0_c_b, s2_b0_proj_wm, s2_b0_proj_s, s2_b0_proj_b, s2_b1_a_wm, s2_b1_a_s, s2_b1_a_b, s2_b1_b_wm, s2_b1_b_s, s2_b1_b_b, s2_b1_c_wm, s2_b1_c_s, s2_b1_c_b, s3_b0_a_wm, s3_b0_a_s, s3_b0_a_b, s3_b0_b_wm, s3_b0_b_s, s3_b0_b_b, s3_b0_c_wm, s3_b0_c_s, s3_b0_c_b, s3_b0_proj_wm, s3_b0_proj_s, s3_b0_proj_b, s3_b1_a_wm, s3_b1_a_s, s3_b1_a_b, s3_b1_b_wm, s3_b1_b_s, s3_b1_b_b, s3_b1_c_wm, s3_b1_c_s, s3_b1_c_b, s4_b0_a_wm, s4_b0_a_s, s4_b0_a_b, s4_b0_b_wm, s4_b0_b_s, s4_b0_b_b, s4_b0_c_wm, s4_b0_c_s, s4_b0_c_b, s4_b0_proj_wm, s4_b0_proj_s, s4_b0_proj_b, s4_b1_a_wm, s4_b1_a_s, s4_b1_a_b, s4_b1_b_wm, s4_b1_b_s, s4_b1_b_b, s4_b1_c_wm, s4_b1_c_s, s4_b1_c_b, s5_b0_a_wm, s5_b0_a_s, s5_b0_a_b, s5_b0_b_wm, s5_b0_b_s, s5_b0_b_b, s5_b0_c_wm, s5_b0_c_s, s5_b0_c_b, s5_b0_proj_wm, s5_b0_proj_s, s5_b0_proj_b, s5_b1_a_wm, s5_b1_a_s, s5_b1_a_b, s5_b1_b_wm, s5_b1_b_s, s5_b1_b_b, s5_b1_c_wm, s5_b1_c_s, s5_b1_c_b, head_w, head_s, head_b)

    x_spec = pl.BlockSpec((_B2, 28, 28, 8, 8), lambda i: (i, 0, 0, 0, 0))
    w_specs = [pl.BlockSpec(a.shape, lambda i, _nd=a.ndim: (0,) * _nd)
               for a in args]

    out = pl.pallas_call(
        _stages_kernel,
        out_shape=jax.ShapeDtypeStruct((n // _B2, _B2, ncls), jnp.float32),
        grid=(n // _B2,),
        in_specs=[x_spec] + w_specs,
        out_specs=pl.BlockSpec((1, _B2, ncls), lambda i: (i, 0, 0)),
        compiler_params=pltpu.CompilerParams(
            dimension_semantics=("parallel",),
            vmem_limit_bytes=56 * 1024 * 1024),
    )(h1, *args)
    return out.reshape(n, ncls)
